# baseline (speedup 1.0000x reference)
import functools

import jax
import jax.numpy as jnp
from jax import lax
from jax.experimental import pallas as pl
from jax.experimental.pallas import tpu as pltpu
from jax.experimental.pallas import tpu_sc as plsc

N = 10000
DF = 128
H = 8
DH = 16

_ROWS = 1000

_NC = 2
_NS = 16
_NW = _NC * _NS
_NP = 10112
_RT = _NP // _NS
_CH = 64
_NCHUNK = 160
_EPT = _CH * _NCHUNK
_EP = _EPT * _NW
_AW = DF + H


def _proj_body(h_ref, w_ref, al_ref, ar_ref, feat_ref, el_ref, er_ref):
    f = jnp.dot(h_ref[...], w_ref[...], preferred_element_type=jnp.float32)
    feat_ref[...] = f
    el_ref[...] = jnp.dot(f, al_ref[...], preferred_element_type=jnp.float32)
    er_ref[...] = jnp.dot(f, ar_ref[...], preferred_element_type=jnp.float32)


_PROJ_OUT = [
    jax.ShapeDtypeStruct((_NP, DF), jnp.float32),
    jax.ShapeDtypeStruct((_NP, 2 * H), jnp.float32),
    jax.ShapeDtypeStruct((_NP, 2 * H), jnp.float32),
]
_PROJ_OUT_SPECS = [
    pl.BlockSpec((_ROWS, DF), lambda i: (i, 0)),
    pl.BlockSpec((_ROWS, 2 * H), lambda i: (i, 0)),
    pl.BlockSpec((_ROWS, 2 * H), lambda i: (i, 0)),
]


def _proj(h, W_i, AL, AR):
    grid = (N // _ROWS,)
    return pl.pallas_call(
        _proj_body,
        grid=grid,
        in_specs=[
            pl.BlockSpec((_ROWS, DF), lambda i: (i, 0)),
            pl.BlockSpec((DF, DF), lambda i: (0, 0)),
            pl.BlockSpec((DF, 2 * H), lambda i: (0, 0)),
            pl.BlockSpec((DF, 2 * H), lambda i: (0, 0)),
        ],
        out_specs=_PROJ_OUT_SPECS,
        out_shape=_PROJ_OUT,
    )(h, W_i, AL, AR)


def _post_math(rstp_ref, denp_ref, sel_ref, bias_ref, ac_ref, g1_ref,
               b1n_ref, g2_ref, b2n_ref, w1_ref, bf1_ref, w2_ref, bf2_ref,
               af_ref):
    rst = rstp_ref[0] + rstp_ref[1]
    den = denp_ref[0] + denp_ref[1]
    den_b = jnp.dot(den, sel_ref[...], preferred_element_type=jnp.float32)
    y = rst / (den_b + 1e-9)
    y = y + bias_ref[...]
    ac = ac_ref[0, 0]
    y = jnp.where(y >= 0, y, ac * y)
    mu = jnp.mean(y, axis=-1, keepdims=True)
    var = jnp.mean((y - mu) ** 2, axis=-1, keepdims=True)
    h1 = (y - mu) * lax.rsqrt(var + 1e-5) * g1_ref[...] + b1n_ref[...]
    mu2 = jnp.mean(h1, axis=-1, keepdims=True)
    var2 = jnp.mean((h1 - mu2) ** 2, axis=-1, keepdims=True)
    hn = (h1 - mu2) * lax.rsqrt(var2 + 1e-5) * g2_ref[...] + b2n_ref[...]
    t = jnp.dot(hn, w1_ref[...], preferred_element_type=jnp.float32) + bf1_ref[...]
    af = af_ref[0, 0]
    t = jnp.where(t >= 0, t, af * t)
    t = jnp.dot(t, w2_ref[...], preferred_element_type=jnp.float32) + bf2_ref[...]
    return h1 + t


def _post_body(rstp_ref, denp_ref, sel_ref, bias_ref, ac_ref, g1_ref,
               b1n_ref, g2_ref, b2n_ref, w1_ref, bf1_ref, w2_ref, bf2_ref,
               af_ref, out_ref):
    out_ref[...] = _post_math(rstp_ref, denp_ref, sel_ref, bias_ref, ac_ref,
                              g1_ref, b1n_ref, g2_ref, b2n_ref, w1_ref,
                              bf1_ref, w2_ref, bf2_ref, af_ref)


def _postproj_body(rstp_ref, denp_ref, sel_ref, bias_ref, ac_ref, g1_ref,
                   b1n_ref, g2_ref, b2n_ref, w1_ref, bf1_ref, w2_ref,
                   bf2_ref, af_ref, wn_ref, aln_ref, arn_ref, out_ref,
                   feat_ref, el_ref, er_ref):
    h = _post_math(rstp_ref, denp_ref, sel_ref, bias_ref, ac_ref, g1_ref,
                   b1n_ref, g2_ref, b2n_ref, w1_ref, bf1_ref, w2_ref,
                   bf2_ref, af_ref)
    out_ref[...] = h
    _proj_body(out_ref, wn_ref, aln_ref, arn_ref, feat_ref, el_ref, er_ref)


_POST_IN_SPECS = [
    pl.BlockSpec((2, _ROWS, DF), lambda i: (0, i, 0)),
    pl.BlockSpec((2, _ROWS, 2 * H), lambda i: (0, i, 0)),
    pl.BlockSpec((2 * H, DF), lambda i: (0, 0)),
    pl.BlockSpec((1, DF), lambda i: (0, 0)),
    pl.BlockSpec((1, 1), lambda i: (0, 0), memory_space=pltpu.SMEM),
    pl.BlockSpec((1, DF), lambda i: (0, 0)),
    pl.BlockSpec((1, DF), lambda i: (0, 0)),
    pl.BlockSpec((1, DF), lambda i: (0, 0)),
    pl.BlockSpec((1, DF), lambda i: (0, 0)),
    pl.BlockSpec((DF, DF), lambda i: (0, 0)),
    pl.BlockSpec((1, DF), lambda i: (0, 0)),
    pl.BlockSpec((DF, DF), lambda i: (0, 0)),
    pl.BlockSpec((1, DF), lambda i: (0, 0)),
    pl.BlockSpec((1, 1), lambda i: (0, 0), memory_space=pltpu.SMEM),
]


def _post(rstp, denp, *args):
    grid = (N // _ROWS,)
    return pl.pallas_call(
        _post_body,
        grid=grid,
        in_specs=_POST_IN_SPECS,
        out_specs=pl.BlockSpec((_ROWS, DF), lambda i: (i, 0)),
        out_shape=jax.ShapeDtypeStruct((N, DF), jnp.float32),
    )(rstp, denp, *args)


def _postproj(rstp, denp, *args):
    grid = (N // _ROWS,)
    return pl.pallas_call(
        _postproj_body,
        grid=grid,
        in_specs=_POST_IN_SPECS + [
            pl.BlockSpec((DF, DF), lambda i: (0, 0)),
            pl.BlockSpec((DF, 2 * H), lambda i: (0, 0)),
            pl.BlockSpec((DF, 2 * H), lambda i: (0, 0)),
        ],
        out_specs=[pl.BlockSpec((_ROWS, DF), lambda i: (i, 0))]
        + _PROJ_OUT_SPECS,
        out_shape=[jax.ShapeDtypeStruct((N, DF), jnp.float32)] + _PROJ_OUT,
    )(rstp, denp, *args)


_NSEG = 2
_SEG = _NCHUNK // _NSEG


def _sc_edge_body(feat_hbm, el_hbm, er_hbm, src_hbm, dst_hbm,
                  rst_out, den_out,
                  srcseg, dstseg, ela, era, feata, wba, elb, erb, featb, wbb,
                  semfa, semla, semra, semfb, semlb, semrb,
                  semra_s, semda_s, semrb_s, semdb_s,
                  shared_rst, shared_den):
    cid = lax.axis_index("c")
    sid = lax.axis_index("s")
    wid = sid * _NC + cid

    zeros16 = jnp.zeros((16,), jnp.float32)

    def _zero_rows(c, _):
        for j in range(DF // 16):
            feata[c, pl.ds(j * 16, 16)] = zeros16
        wba[c, :] = zeros16
        return 0
    lax.fori_loop(0, _CH, _zero_rows, 0)

    rbase = sid * _RT
    zcps = []
    for k in range(_RT // _CH):
        zcps.append(pltpu.async_copy(
            feata, shared_rst.at[pl.ds(rbase + k * _CH, _CH)], semfa))
        zcps.append(pltpu.async_copy(
            wba, shared_den.at[pl.ds(rbase + k * _CH, _CH)], semla))
    rem = _RT % _CH
    if rem:
        zcps.append(pltpu.async_copy(
            feata.at[pl.ds(0, rem)],
            shared_rst.at[pl.ds(rbase + (_RT // _CH) * _CH, rem)], semfa))
        zcps.append(pltpu.async_copy(
            wba.at[pl.ds(0, rem)],
            shared_den.at[pl.ds(rbase + (_RT // _CH) * _CH, rem)], semla))
    for cp in zcps:
        cp.wait()
    plsc.subcore_barrier()

    dn = lax.GatherDimensionNumbers(
        offset_dims=(), collapsed_slice_dims=(0,), start_index_map=(0,))

    def _start_gathers(ci, elx, erx, featx, semf, seml, semr):
        pltpu.async_copy(feat_hbm.at[srcseg.at[ci]], featx, semf)
        pltpu.async_copy(el_hbm.at[srcseg.at[ci]], elx, seml)
        pltpu.async_copy(er_hbm.at[dstseg.at[ci]], erx, semr)

    def _wait_gathers(ci, elx, erx, featx, semf, seml, semr):
        pltpu.make_async_copy(feat_hbm.at[srcseg.at[ci]], featx, semf).wait()
        pltpu.make_async_copy(el_hbm.at[srcseg.at[ci]], elx, seml).wait()
        pltpu.make_async_copy(er_hbm.at[dstseg.at[ci]], erx, semr).wait()

    def _compute(elx, erx, featx, wbx):
        @plsc.parallel_loop(0, _CH, 1, unroll=8)
        def _edge(c):
            e = elx[c, :] + erx[c, :]
            e = jnp.where(e >= 0.0, e, 0.2 * e)
            w = jnp.exp(e)
            wbx[c, :] = w
            for h in range(H):
                wv = lax.gather(
                    w, jnp.full((16, 1), h, jnp.int32), dn, (1,),
                    mode=lax.GatherScatterMode.PROMISE_IN_BOUNDS)
                f = featx[c, pl.ds(h * DH, DH)]
                featx[c, pl.ds(h * DH, DH)] = f * wv

    def _wait_scatters(ci, featx, wbx, semr_s, semd_s):
        pltpu.make_async_copy(
            featx, shared_rst.at[dstseg.at[ci]], semr_s).wait()
        pltpu.make_async_copy(
            wbx, shared_den.at[dstseg.at[ci]], semd_s).wait()

    def _start_scatters(ci, featx, wbx, semr_s, semd_s):
        pltpu.async_copy(featx, shared_rst.at[dstseg.at[ci]], semr_s,
                         add=True)
        pltpu.async_copy(wbx, shared_den.at[dstseg.at[ci]], semd_s, add=True)

    npair = _SEG // 2
    for seg in range(_NSEG):
        pltpu.sync_copy(src_hbm.at[wid, seg], srcseg)
        pltpu.sync_copy(dst_hbm.at[wid, seg], dstseg)

        def _pair(i, _):
            c0 = 2 * i
            c1 = 2 * i + 1
            _start_gathers(c1, elb, erb, featb, semfb, semlb, semrb)
            _wait_gathers(c0, ela, era, feata, semfa, semla, semra)

            @pl.when(i > 0)
            def _():
                _wait_scatters(c0, feata, wba, semra_s, semda_s)
            _compute(ela, era, feata, wba)
            _start_scatters(c0, feata, wba, semra_s, semda_s)

            @pl.when(i < npair - 1)
            def _():
                _start_gathers(c0 + 2, ela, era, feata, semfa, semla, semra)
            _wait_gathers(c1, elb, erb, featb, semfb, semlb, semrb)

            @pl.when(i > 0)
            def _():
                _wait_scatters(c1, featb, wbb, semrb_s, semdb_s)
            _compute(elb, erb, featb, wbb)
            _start_scatters(c1, featb, wbb, semrb_s, semdb_s)
            return 0

        _start_gathers(0, ela, era, feata, semfa, semla, semra)
        lax.fori_loop(0, npair, _pair, 0)
        _wait_scatters(0, feata, wba, semra_s, semda_s)
        _wait_scatters(0, featb, wbb, semrb_s, semdb_s)
    plsc.subcore_barrier()

    pltpu.sync_copy(shared_rst.at[pl.ds(rbase, _RT)],
                    rst_out.at[cid, pl.ds(rbase, _RT)])
    pltpu.sync_copy(shared_den.at[pl.ds(rbase, _RT)],
                    den_out.at[cid, pl.ds(rbase, _RT)])


@functools.partial(
    pl.kernel,
    out_type=[
        jax.ShapeDtypeStruct((_NC, _NP, DF), jnp.float32),
        jax.ShapeDtypeStruct((_NC, _NP, 2 * H), jnp.float32),
    ],
    mesh=plsc.VectorSubcoreMesh(
        core_axis_name="c", subcore_axis_name="s",
        num_cores=_NC, num_subcores=_NS),
    compiler_params=pltpu.CompilerParams(use_tc_tiling_on_sc=False),
    scratch_types=[
        pltpu.VMEM((_SEG, _CH), jnp.int32),
        pltpu.VMEM((_SEG, _CH), jnp.int32),
        pltpu.VMEM((_CH, 2 * H), jnp.float32),
        pltpu.VMEM((_CH, 2 * H), jnp.float32),
        pltpu.VMEM((_CH, DF), jnp.float32),
        pltpu.VMEM((_CH, 2 * H), jnp.float32),
        pltpu.VMEM((_CH, 2 * H), jnp.float32),
        pltpu.VMEM((_CH, 2 * H), jnp.float32),
        pltpu.VMEM((_CH, DF), jnp.float32),
        pltpu.VMEM((_CH, 2 * H), jnp.float32),
        pltpu.SemaphoreType.DMA,
        pltpu.SemaphoreType.DMA,
        pltpu.SemaphoreType.DMA,
        pltpu.SemaphoreType.DMA,
        pltpu.SemaphoreType.DMA,
        pltpu.SemaphoreType.DMA,
        pltpu.SemaphoreType.DMA,
        pltpu.SemaphoreType.DMA,
        pltpu.SemaphoreType.DMA,
        pltpu.SemaphoreType.DMA,
        pltpu.VMEM_SHARED((_NP, DF), jnp.float32),
        pltpu.VMEM_SHARED((_NP, 2 * H), jnp.float32),
    ],
)
def _sc_edge(feat_hbm, el_hbm, er_hbm, src_hbm, dst_hbm, rst_out, den_out,
             *scratch):
    _sc_edge_body(feat_hbm, el_hbm, er_hbm, src_hbm, dst_hbm, rst_out,
                  den_out, *scratch)


def kernel(x, edge_index, W, attn_l, attn_r, bias, prelu_conv, ln1_g, ln1_b,
           ln2_g, ln2_b, W1, b1, W2, b2, prelu_ff):
    src = edge_index[0]
    dst = edge_index[1]
    npad = _EP - src.shape[0]
    ppt = npad // _NW
    pad_src = (jnp.arange(npad, dtype=jnp.int32) * 131) % N
    pad_dst = N + (jnp.arange(npad, dtype=jnp.int32) % (_NP - N))
    src_p = jnp.concatenate(
        [src.reshape(_NW, -1), pad_src.reshape(_NW, ppt)], axis=1)
    dst_p = jnp.concatenate(
        [dst.reshape(_NW, -1), pad_dst.reshape(_NW, ppt)], axis=1)
    src_p = src_p.reshape(_NW, _NSEG, _SEG, _CH)
    dst_p = dst_p.reshape(_NW, _NSEG, _SEG, _CH)
    L = W.shape[0]
    eye = jnp.eye(H, dtype=jnp.float32)
    blk = jnp.repeat(eye, DH, axis=0)
    SEL = jnp.repeat(eye, DH, axis=1)
    SEL16 = jnp.concatenate([SEL, jnp.zeros((H, DF), jnp.float32)])

    def attn_mats(i):
        AL = blk * attn_l[i].reshape(DF, 1)
        AR = blk * attn_r[i].reshape(DF, 1)
        return (jnp.concatenate([AL, AL], axis=1),
                jnp.concatenate([AR, AR], axis=1))

    def post_args(i):
        return (SEL16, bias[i].reshape(1, DF), prelu_conv[i].reshape(1, 1),
                ln1_g[i].reshape(1, DF), ln1_b[i].reshape(1, DF),
                ln2_g[i].reshape(1, DF), ln2_b[i].reshape(1, DF), W1[i],
                b1[i].reshape(1, DF), W2[i], b2[i].reshape(1, DF),
                prelu_ff[i].reshape(1, 1))

    outs = []
    AL2, AR2 = attn_mats(0)
    feat, el2, er2 = _proj(x, W[0], AL2, AR2)
    for i in range(L):
        rstp, denp = _sc_edge(feat, el2, er2, src_p, dst_p)
        if i + 1 < L:
            AL2, AR2 = attn_mats(i + 1)
            h, feat, el2, er2 = _postproj(rstp, denp, *post_args(i),
                                          W[i + 1], AL2, AR2)
        else:
            h = _post(rstp, denp, *post_args(i))
        outs.append(h)
    return tuple(outs)


# speedup vs baseline: 1.2847x; 1.2847x over previous
import functools

import jax
import jax.numpy as jnp
import numpy as np
from jax import lax
from jax.experimental import pallas as pl
from jax.experimental.pallas import tpu as pltpu
from jax.experimental.pallas import tpu_sc as plsc

N = 10000
DF = 128
H = 8
DH = 16

_ROWS = 1000

_NC = 2
_NS = 16
_NW = _NC * _NS
_NP = 10112
_RT = _NP // _NS
_CH = 64
_NCHUNK = 160
_EPT = _CH * _NCHUNK
_EP = _EPT * _NW
_AW = DF + H


def _proj_body(h_ref, w_ref, al_ref, ar_ref, feat_ref, el_ref, er_ref):
    f = jnp.dot(h_ref[...], w_ref[...], preferred_element_type=jnp.float32)
    feat_ref[...] = f.astype(jnp.bfloat16)
    el_ref[...] = jnp.dot(f, al_ref[...], preferred_element_type=jnp.float32)
    er_ref[...] = jnp.dot(f, ar_ref[...], preferred_element_type=jnp.float32)


_PROJ_OUT = [
    jax.ShapeDtypeStruct((_NP, DF), jnp.bfloat16),
    jax.ShapeDtypeStruct((_NP, 2 * H), jnp.float32),
    jax.ShapeDtypeStruct((_NP, 2 * H), jnp.float32),
]
_PROJ_OUT_SPECS = [
    pl.BlockSpec((_ROWS, DF), lambda i: (i, 0)),
    pl.BlockSpec((_ROWS, 2 * H), lambda i: (i, 0)),
    pl.BlockSpec((_ROWS, 2 * H), lambda i: (i, 0)),
]


def _proj(h, W_i, AL, AR):
    grid = (N // _ROWS,)
    return pl.pallas_call(
        _proj_body,
        grid=grid,
        in_specs=[
            pl.BlockSpec((_ROWS, DF), lambda i: (i, 0)),
            pl.BlockSpec((DF, DF), lambda i: (0, 0)),
            pl.BlockSpec((DF, 2 * H), lambda i: (0, 0)),
            pl.BlockSpec((DF, 2 * H), lambda i: (0, 0)),
        ],
        out_specs=_PROJ_OUT_SPECS,
        out_shape=_PROJ_OUT,
    )(h, W_i, AL, AR)


def _post_math(rstp_ref, denp_ref, sel_ref, bias_ref, ac_ref, g1_ref,
               b1n_ref, g2_ref, b2n_ref, w1_ref, bf1_ref, w2_ref, bf2_ref,
               af_ref, pm_ref):
    rst = rstp_ref[0] + rstp_ref[1]
    den = denp_ref[0] + denp_ref[1]
    den_b = jnp.dot(den, sel_ref[...], preferred_element_type=jnp.float32)
    y = rst / (den_b + 1e-9)
    y = y + bias_ref[...]
    ac = ac_ref[0, 0]
    y = jnp.where(y >= 0, y, ac * y)
    mu = jnp.mean(y, axis=-1, keepdims=True)
    var = jnp.mean((y - mu) ** 2, axis=-1, keepdims=True)
    ynorm = (y - mu) * lax.rsqrt(var + 1e-5)
    ynorm = jnp.dot(ynorm, pm_ref[...], preferred_element_type=jnp.float32)
    h1 = ynorm * g1_ref[...] + b1n_ref[...]
    mu2 = jnp.mean(h1, axis=-1, keepdims=True)
    var2 = jnp.mean((h1 - mu2) ** 2, axis=-1, keepdims=True)
    hn = (h1 - mu2) * lax.rsqrt(var2 + 1e-5) * g2_ref[...] + b2n_ref[...]
    t = jnp.dot(hn, w1_ref[...], preferred_element_type=jnp.float32) + bf1_ref[...]
    af = af_ref[0, 0]
    t = jnp.where(t >= 0, t, af * t)
    t = jnp.dot(t, w2_ref[...], preferred_element_type=jnp.float32) + bf2_ref[...]
    return h1 + t


def _post_body(rstp_ref, denp_ref, sel_ref, bias_ref, ac_ref, g1_ref,
               b1n_ref, g2_ref, b2n_ref, w1_ref, bf1_ref, w2_ref, bf2_ref,
               af_ref, pm_ref, out_ref):
    out_ref[...] = _post_math(rstp_ref, denp_ref, sel_ref, bias_ref, ac_ref,
                              g1_ref, b1n_ref, g2_ref, b2n_ref, w1_ref,
                              bf1_ref, w2_ref, bf2_ref, af_ref, pm_ref)


def _postproj_body(rstp_ref, denp_ref, sel_ref, bias_ref, ac_ref, g1_ref,
                   b1n_ref, g2_ref, b2n_ref, w1_ref, bf1_ref, w2_ref,
                   bf2_ref, af_ref, pm_ref, wn_ref, aln_ref, arn_ref,
                   out_ref, feat_ref, el_ref, er_ref):
    h = _post_math(rstp_ref, denp_ref, sel_ref, bias_ref, ac_ref, g1_ref,
                   b1n_ref, g2_ref, b2n_ref, w1_ref, bf1_ref, w2_ref,
                   bf2_ref, af_ref, pm_ref)
    out_ref[...] = h
    _proj_body(out_ref, wn_ref, aln_ref, arn_ref, feat_ref, el_ref, er_ref)


_POST_IN_SPECS = [
    pl.BlockSpec((2, _ROWS, DF), lambda i: (0, i, 0)),
    pl.BlockSpec((2, _ROWS, 2 * H), lambda i: (0, i, 0)),
    pl.BlockSpec((2 * H, DF), lambda i: (0, 0)),
    pl.BlockSpec((1, DF), lambda i: (0, 0)),
    pl.BlockSpec((1, 1), lambda i: (0, 0), memory_space=pltpu.SMEM),
    pl.BlockSpec((1, DF), lambda i: (0, 0)),
    pl.BlockSpec((1, DF), lambda i: (0, 0)),
    pl.BlockSpec((1, DF), lambda i: (0, 0)),
    pl.BlockSpec((1, DF), lambda i: (0, 0)),
    pl.BlockSpec((DF, DF), lambda i: (0, 0)),
    pl.BlockSpec((1, DF), lambda i: (0, 0)),
    pl.BlockSpec((DF, DF), lambda i: (0, 0)),
    pl.BlockSpec((1, DF), lambda i: (0, 0)),
    pl.BlockSpec((1, 1), lambda i: (0, 0), memory_space=pltpu.SMEM),
    pl.BlockSpec((DF, DF), lambda i: (0, 0)),
]


def _post(rstp, denp, *args):
    grid = (N // _ROWS,)
    return pl.pallas_call(
        _post_body,
        grid=grid,
        in_specs=_POST_IN_SPECS,
        out_specs=pl.BlockSpec((_ROWS, DF), lambda i: (i, 0)),
        out_shape=jax.ShapeDtypeStruct((N, DF), jnp.float32),
    )(rstp, denp, *args)


def _postproj(rstp, denp, *args):
    grid = (N // _ROWS,)
    return pl.pallas_call(
        _postproj_body,
        grid=grid,
        in_specs=_POST_IN_SPECS + [
            pl.BlockSpec((DF, DF), lambda i: (0, 0)),
            pl.BlockSpec((DF, 2 * H), lambda i: (0, 0)),
            pl.BlockSpec((DF, 2 * H), lambda i: (0, 0)),
        ],
        out_specs=[pl.BlockSpec((_ROWS, DF), lambda i: (i, 0))]
        + _PROJ_OUT_SPECS,
        out_shape=[jax.ShapeDtypeStruct((N, DF), jnp.float32)] + _PROJ_OUT,
    )(rstp, denp, *args)


_NSEG = 4
_SEG = _NCHUNK // _NSEG


def _sc_edge_body(feat_hbm, el_hbm, er_hbm, src_hbm, dst_hbm,
                  rst_out, den_out,
                  srcseg, dstseg, ela, era, feata, msga, wba,
                  elb, erb, featb, msgb, wbb,
                  semfa, semla, semra, semfb, semlb, semrb,
                  semra_s, semda_s, semrb_s, semdb_s,
                  shared_rst, shared_den):
    cid = lax.axis_index("c")
    sid = lax.axis_index("s")
    wid = sid * _NC + cid

    zeros16 = jnp.zeros((16,), jnp.float32)

    def _zero_rows(c, _):
        for j in range(DF // 16):
            msga[c, pl.ds(j * 16, 16)] = zeros16
        wba[c, :] = zeros16
        return 0
    lax.fori_loop(0, _CH, _zero_rows, 0)

    rbase = sid * _RT
    zcps = []
    for k in range(_RT // _CH):
        zcps.append(pltpu.async_copy(
            msga, shared_rst.at[pl.ds(rbase + k * _CH, _CH)], semfa))
        zcps.append(pltpu.async_copy(
            wba, shared_den.at[pl.ds(rbase + k * _CH, _CH)], semla))
    rem = _RT % _CH
    if rem:
        zcps.append(pltpu.async_copy(
            msga.at[pl.ds(0, rem)],
            shared_rst.at[pl.ds(rbase + (_RT // _CH) * _CH, rem)], semfa))
        zcps.append(pltpu.async_copy(
            wba.at[pl.ds(0, rem)],
            shared_den.at[pl.ds(rbase + (_RT // _CH) * _CH, rem)], semla))
    for cp in zcps:
        cp.wait()
    plsc.subcore_barrier()

    dn = lax.GatherDimensionNumbers(
        offset_dims=(), collapsed_slice_dims=(0,), start_index_map=(0,))

    def _start_gathers(ci, elx, erx, featx, semf, seml, semr):
        pltpu.async_copy(feat_hbm.at[srcseg.at[ci]], featx, semf)
        pltpu.async_copy(el_hbm.at[srcseg.at[ci]], elx, seml)
        pltpu.async_copy(er_hbm.at[dstseg.at[ci]], erx, semr)

    def _wait_gathers(ci, elx, erx, featx, semf, seml, semr):
        pltpu.make_async_copy(feat_hbm.at[srcseg.at[ci]], featx, semf).wait()
        pltpu.make_async_copy(el_hbm.at[srcseg.at[ci]], elx, seml).wait()
        pltpu.make_async_copy(er_hbm.at[dstseg.at[ci]], erx, semr).wait()

    mask_hi = lax.iota(jnp.int32, 16) >= 8

    def _splat(w, h):
        return lax.gather(w, jnp.full((16, 1), h, jnp.int32), dn, (1,),
                          mode=lax.GatherScatterMode.PROMISE_IN_BOUNDS)

    def _compute(elx, erx, featx, msgx, wbx):
        @plsc.parallel_loop(0, _CH, 1, unroll=4)
        def _edge(c):
            e = elx[c, :] + erx[c, :]
            e = jnp.where(e >= 0.0, e, 0.2 * e)
            w = jnp.exp(e)
            wbx[c, :] = w
            for g in range(4):
                x = featx[c, pl.ds(g * 32, 32)]
                a, b = plsc.unpack(x, format=plsc.PackFormat.INTERLEAVED)
                wv = jnp.where(mask_hi, _splat(w, 2 * g + 1), _splat(w, 2 * g))
                msgx[c, pl.ds(g * 32, 16)] = a * wv
                msgx[c, pl.ds(g * 32 + 16, 16)] = b * wv

    def _wait_scatters(ci, msgx, wbx, semr_s, semd_s):
        pltpu.make_async_copy(
            msgx, shared_rst.at[dstseg.at[ci]], semr_s).wait()
        pltpu.make_async_copy(
            wbx, shared_den.at[dstseg.at[ci]], semd_s).wait()

    def _start_scatters(ci, msgx, wbx, semr_s, semd_s):
        pltpu.async_copy(msgx, shared_rst.at[dstseg.at[ci]], semr_s,
                         add=True)
        pltpu.async_copy(wbx, shared_den.at[dstseg.at[ci]], semd_s, add=True)

    npair = _SEG // 2
    for seg in range(_NSEG):
        pltpu.sync_copy(src_hbm.at[wid, seg], srcseg)
        pltpu.sync_copy(dst_hbm.at[wid, seg], dstseg)

        def _pair(i, _):
            c0 = 2 * i
            c1 = 2 * i + 1
            _start_gathers(c1, elb, erb, featb, semfb, semlb, semrb)
            _wait_gathers(c0, ela, era, feata, semfa, semla, semra)

            @pl.when(i > 0)
            def _():
                _wait_scatters(c0, msga, wba, semra_s, semda_s)
            _compute(ela, era, feata, msga, wba)
            _start_scatters(c0, msga, wba, semra_s, semda_s)

            @pl.when(i < npair - 1)
            def _():
                _start_gathers(c0 + 2, ela, era, feata, semfa, semla, semra)
            _wait_gathers(c1, elb, erb, featb, semfb, semlb, semrb)

            @pl.when(i > 0)
            def _():
                _wait_scatters(c1, msgb, wbb, semrb_s, semdb_s)
            _compute(elb, erb, featb, msgb, wbb)
            _start_scatters(c1, msgb, wbb, semrb_s, semdb_s)
            return 0

        _start_gathers(0, ela, era, feata, semfa, semla, semra)
        lax.fori_loop(0, npair, _pair, 0)
        _wait_scatters(0, msga, wba, semra_s, semda_s)
        _wait_scatters(0, msgb, wbb, semrb_s, semdb_s)
    plsc.subcore_barrier()

    pltpu.sync_copy(shared_rst.at[pl.ds(rbase, _RT)],
                    rst_out.at[cid, pl.ds(rbase, _RT)])
    pltpu.sync_copy(shared_den.at[pl.ds(rbase, _RT)],
                    den_out.at[cid, pl.ds(rbase, _RT)])


@functools.partial(
    pl.kernel,
    out_type=[
        jax.ShapeDtypeStruct((_NC, _NP, DF), jnp.float32),
        jax.ShapeDtypeStruct((_NC, _NP, 2 * H), jnp.float32),
    ],
    mesh=plsc.VectorSubcoreMesh(
        core_axis_name="c", subcore_axis_name="s",
        num_cores=_NC, num_subcores=_NS),
    compiler_params=pltpu.CompilerParams(use_tc_tiling_on_sc=False,
                                         needs_layout_passes=False),
    scratch_types=[
        pltpu.VMEM((_SEG, _CH), jnp.int32),
        pltpu.VMEM((_SEG, _CH), jnp.int32),
        pltpu.VMEM((_CH, 2 * H), jnp.float32),
        pltpu.VMEM((_CH, 2 * H), jnp.float32),
        pltpu.VMEM((_CH, DF), jnp.bfloat16),
        pltpu.VMEM((_CH, DF), jnp.float32),
        pltpu.VMEM((_CH, 2 * H), jnp.float32),
        pltpu.VMEM((_CH, 2 * H), jnp.float32),
        pltpu.VMEM((_CH, 2 * H), jnp.float32),
        pltpu.VMEM((_CH, DF), jnp.bfloat16),
        pltpu.VMEM((_CH, DF), jnp.float32),
        pltpu.VMEM((_CH, 2 * H), jnp.float32),
        pltpu.SemaphoreType.DMA,
        pltpu.SemaphoreType.DMA,
        pltpu.SemaphoreType.DMA,
        pltpu.SemaphoreType.DMA,
        pltpu.SemaphoreType.DMA,
        pltpu.SemaphoreType.DMA,
        pltpu.SemaphoreType.DMA,
        pltpu.SemaphoreType.DMA,
        pltpu.SemaphoreType.DMA,
        pltpu.SemaphoreType.DMA,
        pltpu.VMEM_SHARED((_NP, DF), jnp.float32),
        pltpu.VMEM_SHARED((_NP, 2 * H), jnp.float32),
    ],
)
def _sc_edge(feat_hbm, el_hbm, er_hbm, src_hbm, dst_hbm, rst_out, den_out,
             *scratch):
    _sc_edge_body(feat_hbm, el_hbm, er_hbm, src_hbm, dst_hbm, rst_out,
                  den_out, *scratch)


def kernel(x, edge_index, W, attn_l, attn_r, bias, prelu_conv, ln1_g, ln1_b,
           ln2_g, ln2_b, W1, b1, W2, b2, prelu_ff):
    src = edge_index[0]
    dst = edge_index[1]
    npad = _EP - src.shape[0]
    ppt = npad // _NW
    pad_src = (jnp.arange(npad, dtype=jnp.int32) * 131) % N
    pad_dst = N + (jnp.arange(npad, dtype=jnp.int32) % (_NP - N))
    src_p = jnp.concatenate(
        [src.reshape(_NW, -1), pad_src.reshape(_NW, ppt)], axis=1)
    dst_p = jnp.concatenate(
        [dst.reshape(_NW, -1), pad_dst.reshape(_NW, ppt)], axis=1)
    src_p = src_p.reshape(_NW, _NSEG, _SEG, _CH)
    dst_p = dst_p.reshape(_NW, _NSEG, _SEG, _CH)
    L = W.shape[0]
    eye = jnp.eye(H, dtype=jnp.float32)
    blk = jnp.repeat(eye, DH, axis=0)
    perm = np.empty((DF,), np.int32)
    for g in range(4):
        perm[g * 32:g * 32 + 16] = g * 32 + 2 * np.arange(16)
        perm[g * 32 + 16:g * 32 + 32] = g * 32 + 2 * np.arange(16) + 1
    SELP = jnp.asarray(
        (perm[None, :] // DH == np.arange(2 * H)[:, None]).astype(np.float32))
    PM = jnp.asarray(
        (perm[:, None] == np.arange(DF)[None, :]).astype(np.float32))
    permj = jnp.asarray(perm)

    def attn_mats(i):
        AL = blk * attn_l[i].reshape(DF, 1)
        AR = blk * attn_r[i].reshape(DF, 1)
        return (jnp.concatenate([AL, AL], axis=1),
                jnp.concatenate([AR, AR], axis=1))

    def post_args(i):
        return (SELP, bias[i].reshape(DF)[permj].reshape(1, DF),
                prelu_conv[i].reshape(1, 1),
                ln1_g[i].reshape(1, DF), ln1_b[i].reshape(1, DF),
                ln2_g[i].reshape(1, DF), ln2_b[i].reshape(1, DF), W1[i],
                b1[i].reshape(1, DF), W2[i], b2[i].reshape(1, DF),
                prelu_ff[i].reshape(1, 1), PM)

    outs = []
    AL2, AR2 = attn_mats(0)
    feat, el2, er2 = _proj(x, W[0], AL2, AR2)
    for i in range(L):
        rstp, denp = _sc_edge(feat, el2, er2, src_p, dst_p)
        if i + 1 < L:
            AL2, AR2 = attn_mats(i + 1)
            h, feat, el2, er2 = _postproj(rstp, denp, *post_args(i),
                                          W[i + 1], AL2, AR2)
        else:
            h = _post(rstp, denp, *post_args(i))
        outs.append(h)
    return tuple(outs)
